# baseline (speedup 1.0000x reference)
import functools

import jax
import jax.numpy as jnp
from jax import lax
from jax.experimental import pallas as pl
from jax.experimental.pallas import tpu as pltpu
from jax.experimental.pallas import tpu_sc as plsc

N = 10000
E = 320000
D = 128

NC = 2
NS = 16
NW = NC * NS

CHUNK = 128
CH0 = 40
CH1 = 120
EP = NS * (CH0 + CH1) * CHUNK
NPAD = 10112
ROWS_PER_S = NPAD // NS

BLK = 8
NBUF = 2
GPB = BLK // NBUF


def _mm_body(feat_ref, w_ref, b_ref, o_ref):
  o_ref[...] = (
      jnp.dot(feat_ref[...], w_ref[...], preferred_element_type=jnp.float32)
      + b_ref[...]
  )


def _add_body(a_ref, b_ref, o_ref):
  o_ref[...] = a_ref[...] + b_ref[...]


_sc_mesh = plsc.VectorSubcoreMesh(core_axis_name="c", subcore_axis_name="s")


@functools.partial(
    pl.kernel,
    out_type=jax.ShapeDtypeStruct((NC, NPAD, D), jnp.float32),
    mesh=_sc_mesh,
    scratch_types=[
        pltpu.VMEM((2, BLK, CHUNK), jnp.int32),
        pltpu.VMEM((2, BLK, CHUNK), jnp.int32),
        pltpu.VMEM((NBUF, CHUNK, D), jnp.float32),
        pltpu.VMEM_SHARED((NPAD, D), jnp.float32),
        pltpu.SemaphoreType.DMA((NBUF,)),
        pltpu.SemaphoreType.DMA((NBUF,)),
        pltpu.SemaphoreType.DMA,
    ],
)
def _sc_aggregate(src_hbm, dst_hbm, h_hbm, z_hbm, out_hbm,
                  sidx, didx, rows_v, acc, gsem, ssem, isem):
  c = lax.axis_index("c")
  s = lax.axis_index("s")

  pltpu.sync_copy(z_hbm.at[pl.ds(s * ROWS_PER_S, ROWS_PER_S)],
                  acc.at[pl.ds(s * ROWS_PER_S, ROWS_PER_S)])
  plsc.subcore_barrier()

  def fire_gather(slot, row, b):
    pltpu.async_copy(h_hbm.at[sidx.at[slot, row]], rows_v.at[b], gsem.at[b])

  def drain_gather(b):
    pltpu.make_async_copy(h_hbm.at[pl.ds(0, CHUNK)], rows_v.at[b],
                          gsem.at[b]).wait()

  def fire_scatter(slot, row, b):
    pltpu.async_copy(rows_v.at[b], acc.at[didx.at[slot, row]], ssem.at[b],
                     add=True)

  def drain_scatter(b):
    pltpu.make_async_copy(h_hbm.at[pl.ds(0, CHUNK)], rows_v.at[b],
                          ssem.at[b]).wait()

  def run_pipeline(row0, nblk):
    def fire_idx(blk, slot):
      off = row0 + blk * BLK
      pltpu.async_copy(src_hbm.at[pl.ds(off, BLK)], sidx.at[slot], isem)
      pltpu.async_copy(dst_hbm.at[pl.ds(off, BLK)], didx.at[slot], isem)

    def drain_idx():
      pltpu.make_async_copy(src_hbm.at[pl.ds(0, BLK)], sidx.at[0], isem).wait()
      pltpu.make_async_copy(src_hbm.at[pl.ds(0, BLK)], didx.at[0], isem).wait()

    pltpu.sync_copy(src_hbm.at[pl.ds(row0, BLK)], sidx.at[0])
    pltpu.sync_copy(dst_hbm.at[pl.ds(row0, BLK)], didx.at[0])

    fire_idx(1, 1)
    for b in range(NBUF):
      fire_gather(0, b, b)

    def block_body(blk, p, fire_next, fire_idx_next):
      q = 1 - p
      for gg in range(GPB):
        r0 = NBUF * gg
        for b in range(NBUF):
          drain_gather(b)
          fire_scatter(p, r0 + b, b)
        if gg == GPB - 1 and (fire_next or fire_idx_next):
          drain_idx()
        for b in range(NBUF):
          drain_scatter(b)
          if gg < GPB - 1:
            fire_gather(p, r0 + NBUF + b, b)
          elif fire_next:
            fire_gather(q, b, b)
      if fire_idx_next:
        fire_idx(blk + 2, p)

    def body(blk, carry):
      p = lax.rem(blk, 2)

      @pl.when(blk < nblk - 2)
      def _steady():
        block_body(blk, p, True, True)

      @pl.when(blk == nblk - 2)
      def _penultimate():
        block_body(blk, p, True, False)

      return carry

    lax.fori_loop(0, nblk - 1, body, 0)
    block_body(nblk - 1, (nblk - 1) % 2, False, False)

  @pl.when(c == 0)
  def _core0():
    run_pipeline(s * CH0, CH0 // BLK)

  @pl.when(c == 1)
  def _core1():
    run_pipeline(NS * CH0 + s * CH1, CH1 // BLK)

  plsc.subcore_barrier()
  pltpu.sync_copy(acc.at[pl.ds(s * ROWS_PER_S, ROWS_PER_S)],
                  out_hbm.at[c, pl.ds(s * ROWS_PER_S, ROWS_PER_S)])


def kernel(feat, edge_index, W, b):
  src = edge_index[0].astype(jnp.int32)
  dst = edge_index[1].astype(jnp.int32)
  pad = EP - E
  srcp = jnp.concatenate([src, jnp.zeros((pad,), jnp.int32)]).reshape(-1, CHUNK)
  dstp = jnp.concatenate([dst, jnp.full((pad,), N, jnp.int32)]).reshape(-1, CHUNK)

  h = pl.pallas_call(
      _mm_body,
      grid=(10,),
      in_specs=[
          pl.BlockSpec((N // 10, D), lambda i: (i, 0)),
          pl.BlockSpec((D, D), lambda i: (0, 0)),
          pl.BlockSpec((1, D), lambda i: (0, 0)),
      ],
      out_specs=pl.BlockSpec((N // 10, D), lambda i: (i, 0)),
      out_shape=jax.ShapeDtypeStruct((N, D), jnp.float32),
  )(feat, W, b.reshape(1, D))

  zeros = jnp.zeros((NPAD, D), jnp.float32)
  partials = _sc_aggregate(srcp, dstp, h, zeros)

  out = pl.pallas_call(
      _add_body,
      grid=(10,),
      in_specs=[
          pl.BlockSpec((N // 10, D), lambda i: (i, 0)),
          pl.BlockSpec((N // 10, D), lambda i: (i, 0)),
      ],
      out_specs=pl.BlockSpec((N // 10, D), lambda i: (i, 0)),
      out_shape=jax.ShapeDtypeStruct((N, D), jnp.float32),
  )(partials[0, :N], partials[1, :N])
  return out


# speedup vs baseline: 1.1662x; 1.1662x over previous
import functools

import jax
import jax.numpy as jnp
from jax import lax
from jax.experimental import pallas as pl
from jax.experimental.pallas import tpu as pltpu
from jax.experimental.pallas import tpu_sc as plsc

N = 10000
E = 320000
D = 128

NC = 2
NS = 16
NW = NC * NS

CHUNK = 128
CH0 = 120
CH1 = 40
EP = NS * (CH0 + CH1) * CHUNK
NPAD = 10112
ROWS_PER_S = NPAD // NS

BLK = 8
NBUF = 2
GPB = BLK // NBUF


def _mm_body(feat_ref, w_ref, b_ref, o_ref):
  o_ref[...] = (
      jnp.dot(feat_ref[...], w_ref[...], preferred_element_type=jnp.float32)
      + b_ref[...]
  )


def _add_body(a_ref, b_ref, o_ref):
  o_ref[...] = a_ref[...] + b_ref[...]


_sc_mesh = plsc.VectorSubcoreMesh(core_axis_name="c", subcore_axis_name="s")


@functools.partial(
    pl.kernel,
    out_type=jax.ShapeDtypeStruct((NC, NPAD, D), jnp.float32),
    mesh=_sc_mesh,
    scratch_types=[
        pltpu.VMEM((2, BLK, CHUNK), jnp.int32),
        pltpu.VMEM((2, BLK, CHUNK), jnp.int32),
        pltpu.VMEM((NBUF, CHUNK, D), jnp.float32),
        pltpu.VMEM_SHARED((NPAD, D), jnp.float32),
        pltpu.SemaphoreType.DMA((NBUF,)),
        pltpu.SemaphoreType.DMA((NBUF,)),
        pltpu.SemaphoreType.DMA,
    ],
)
def _sc_aggregate(src_hbm, dst_hbm, h_hbm, z_hbm, out_hbm,
                  sidx, didx, rows_v, acc, gsem, ssem, isem):
  c = lax.axis_index("c")
  s = lax.axis_index("s")

  pltpu.sync_copy(z_hbm.at[pl.ds(s * ROWS_PER_S, ROWS_PER_S)],
                  acc.at[pl.ds(s * ROWS_PER_S, ROWS_PER_S)])
  plsc.subcore_barrier()

  def fire_gather(slot, row, b):
    pltpu.async_copy(h_hbm.at[sidx.at[slot, row]], rows_v.at[b], gsem.at[b])

  def drain_gather(b):
    pltpu.make_async_copy(h_hbm.at[pl.ds(0, CHUNK)], rows_v.at[b],
                          gsem.at[b]).wait()

  def fire_scatter(slot, row, b):
    pltpu.async_copy(rows_v.at[b], acc.at[didx.at[slot, row]], ssem.at[b],
                     add=True)

  def drain_scatter(b):
    pltpu.make_async_copy(h_hbm.at[pl.ds(0, CHUNK)], rows_v.at[b],
                          ssem.at[b]).wait()

  def run_pipeline(row0, nblk):
    def fire_idx(blk, slot):
      off = row0 + blk * BLK
      pltpu.async_copy(src_hbm.at[pl.ds(off, BLK)], sidx.at[slot], isem)
      pltpu.async_copy(dst_hbm.at[pl.ds(off, BLK)], didx.at[slot], isem)

    def drain_idx():
      pltpu.make_async_copy(src_hbm.at[pl.ds(0, BLK)], sidx.at[0], isem).wait()
      pltpu.make_async_copy(src_hbm.at[pl.ds(0, BLK)], didx.at[0], isem).wait()

    pltpu.sync_copy(src_hbm.at[pl.ds(row0, BLK)], sidx.at[0])
    pltpu.sync_copy(dst_hbm.at[pl.ds(row0, BLK)], didx.at[0])

    fire_idx(1, 1)
    for b in range(NBUF):
      fire_gather(0, b, b)

    def block_body(blk, p, fire_next, fire_idx_next):
      q = 1 - p
      for gg in range(GPB):
        r0 = NBUF * gg
        for b in range(NBUF):
          drain_gather(b)
          fire_scatter(p, r0 + b, b)
        if gg == GPB - 1 and (fire_next or fire_idx_next):
          drain_idx()
        for b in range(NBUF):
          drain_scatter(b)
          if gg < GPB - 1:
            fire_gather(p, r0 + NBUF + b, b)
          elif fire_next:
            fire_gather(q, b, b)
      if fire_idx_next:
        fire_idx(blk + 2, p)

    def body(blk, carry):
      p = lax.rem(blk, 2)

      @pl.when(blk < nblk - 2)
      def _steady():
        block_body(blk, p, True, True)

      @pl.when(blk == nblk - 2)
      def _penultimate():
        block_body(blk, p, True, False)

      return carry

    lax.fori_loop(0, nblk - 1, body, 0)
    block_body(nblk - 1, (nblk - 1) % 2, False, False)

  @pl.when(c == 0)
  def _core0():
    run_pipeline(s * CH0, CH0 // BLK)

  @pl.when(c == 1)
  def _core1():
    run_pipeline(NS * CH0 + s * CH1, CH1 // BLK)

  plsc.subcore_barrier()
  pltpu.sync_copy(acc.at[pl.ds(s * ROWS_PER_S, ROWS_PER_S)],
                  out_hbm.at[c, pl.ds(s * ROWS_PER_S, ROWS_PER_S)])


def kernel(feat, edge_index, W, b):
  src = edge_index[0].astype(jnp.int32)
  dst = edge_index[1].astype(jnp.int32)
  pad = EP - E
  srcp = jnp.concatenate([src, jnp.zeros((pad,), jnp.int32)]).reshape(-1, CHUNK)
  dstp = jnp.concatenate([dst, jnp.full((pad,), N, jnp.int32)]).reshape(-1, CHUNK)

  h = pl.pallas_call(
      _mm_body,
      grid=(10,),
      in_specs=[
          pl.BlockSpec((N // 10, D), lambda i: (i, 0)),
          pl.BlockSpec((D, D), lambda i: (0, 0)),
          pl.BlockSpec((1, D), lambda i: (0, 0)),
      ],
      out_specs=pl.BlockSpec((N // 10, D), lambda i: (i, 0)),
      out_shape=jax.ShapeDtypeStruct((N, D), jnp.float32),
  )(feat, W, b.reshape(1, D))

  zeros = jnp.zeros((NPAD, D), jnp.float32)
  partials = _sc_aggregate(srcp, dstp, h, zeros)

  out = pl.pallas_call(
      _add_body,
      grid=(10,),
      in_specs=[
          pl.BlockSpec((N // 10, D), lambda i: (i, 0)),
          pl.BlockSpec((N // 10, D), lambda i: (i, 0)),
      ],
      out_specs=pl.BlockSpec((N // 10, D), lambda i: (i, 0)),
      out_shape=jax.ShapeDtypeStruct((N, D), jnp.float32),
  )(partials[0, :N], partials[1, :N])
  return out


# speedup vs baseline: 1.2985x; 1.1134x over previous
import functools

import jax
import jax.numpy as jnp
from jax import lax
from jax.experimental import pallas as pl
from jax.experimental.pallas import tpu as pltpu
from jax.experimental.pallas import tpu_sc as plsc

N = 10000
E = 320000
D = 128

NC = 2
NS = 16
NW = NC * NS

CHUNK = 128
CH0 = 144
CH1 = 16
EP = NS * (CH0 + CH1) * CHUNK
NPAD = 10112
ROWS_PER_S = NPAD // NS

BLK = 8
NBUF = 2
GPB = BLK // NBUF


def _mm_body(feat_ref, w_ref, b_ref, o_ref):
  o_ref[...] = (
      jnp.dot(feat_ref[...], w_ref[...], preferred_element_type=jnp.float32)
      + b_ref[...]
  )


def _add_body(a_ref, b_ref, o_ref):
  o_ref[...] = a_ref[...] + b_ref[...]


_sc_mesh = plsc.VectorSubcoreMesh(core_axis_name="c", subcore_axis_name="s")


@functools.partial(
    pl.kernel,
    out_type=jax.ShapeDtypeStruct((NC, NPAD, D), jnp.float32),
    mesh=_sc_mesh,
    scratch_types=[
        pltpu.VMEM((2, BLK, CHUNK), jnp.int32),
        pltpu.VMEM((2, BLK, CHUNK), jnp.int32),
        pltpu.VMEM((NBUF, CHUNK, D), jnp.float32),
        pltpu.VMEM_SHARED((NPAD, D), jnp.float32),
        pltpu.SemaphoreType.DMA((NBUF,)),
        pltpu.SemaphoreType.DMA((NBUF,)),
        pltpu.SemaphoreType.DMA,
    ],
)
def _sc_aggregate(src_hbm, dst_hbm, h_hbm, z_hbm, out_hbm,
                  sidx, didx, rows_v, acc, gsem, ssem, isem):
  c = lax.axis_index("c")
  s = lax.axis_index("s")

  pltpu.sync_copy(z_hbm.at[pl.ds(s * ROWS_PER_S, ROWS_PER_S)],
                  acc.at[pl.ds(s * ROWS_PER_S, ROWS_PER_S)])
  plsc.subcore_barrier()

  def fire_gather(slot, row, b):
    pltpu.async_copy(h_hbm.at[sidx.at[slot, row]], rows_v.at[b], gsem.at[b])

  def drain_gather(b):
    pltpu.make_async_copy(h_hbm.at[pl.ds(0, CHUNK)], rows_v.at[b],
                          gsem.at[b]).wait()

  def fire_scatter(slot, row, b):
    pltpu.async_copy(rows_v.at[b], acc.at[didx.at[slot, row]], ssem.at[b],
                     add=True)

  def drain_scatter(b):
    pltpu.make_async_copy(h_hbm.at[pl.ds(0, CHUNK)], rows_v.at[b],
                          ssem.at[b]).wait()

  def run_pipeline(row0, nblk):
    def fire_idx(blk, slot):
      off = row0 + blk * BLK
      pltpu.async_copy(src_hbm.at[pl.ds(off, BLK)], sidx.at[slot], isem)
      pltpu.async_copy(dst_hbm.at[pl.ds(off, BLK)], didx.at[slot], isem)

    def drain_idx():
      pltpu.make_async_copy(src_hbm.at[pl.ds(0, BLK)], sidx.at[0], isem).wait()
      pltpu.make_async_copy(src_hbm.at[pl.ds(0, BLK)], didx.at[0], isem).wait()

    pltpu.sync_copy(src_hbm.at[pl.ds(row0, BLK)], sidx.at[0])
    pltpu.sync_copy(dst_hbm.at[pl.ds(row0, BLK)], didx.at[0])

    fire_idx(1, 1)
    for b in range(NBUF):
      fire_gather(0, b, b)

    def block_body(blk, p, fire_next, fire_idx_next):
      q = 1 - p
      for gg in range(GPB):
        r0 = NBUF * gg
        for b in range(NBUF):
          drain_gather(b)
          fire_scatter(p, r0 + b, b)
        if gg == GPB - 1 and (fire_next or fire_idx_next):
          drain_idx()
        for b in range(NBUF):
          drain_scatter(b)
          if gg < GPB - 1:
            fire_gather(p, r0 + NBUF + b, b)
          elif fire_next:
            fire_gather(q, b, b)
      if fire_idx_next:
        fire_idx(blk + 2, p)

    def body(blk, carry):
      p = lax.rem(blk, 2)

      @pl.when(blk < nblk - 2)
      def _steady():
        block_body(blk, p, True, True)

      @pl.when(blk == nblk - 2)
      def _penultimate():
        block_body(blk, p, True, False)

      return carry

    lax.fori_loop(0, nblk - 1, body, 0)
    block_body(nblk - 1, (nblk - 1) % 2, False, False)

  @pl.when(c == 0)
  def _core0():
    run_pipeline(s * CH0, CH0 // BLK)

  @pl.when(c == 1)
  def _core1():
    run_pipeline(NS * CH0 + s * CH1, CH1 // BLK)

  plsc.subcore_barrier()
  pltpu.sync_copy(acc.at[pl.ds(s * ROWS_PER_S, ROWS_PER_S)],
                  out_hbm.at[c, pl.ds(s * ROWS_PER_S, ROWS_PER_S)])


def kernel(feat, edge_index, W, b):
  src = edge_index[0].astype(jnp.int32)
  dst = edge_index[1].astype(jnp.int32)
  pad = EP - E
  srcp = jnp.concatenate([src, jnp.zeros((pad,), jnp.int32)]).reshape(-1, CHUNK)
  dstp = jnp.concatenate([dst, jnp.full((pad,), N, jnp.int32)]).reshape(-1, CHUNK)

  h = pl.pallas_call(
      _mm_body,
      grid=(10,),
      in_specs=[
          pl.BlockSpec((N // 10, D), lambda i: (i, 0)),
          pl.BlockSpec((D, D), lambda i: (0, 0)),
          pl.BlockSpec((1, D), lambda i: (0, 0)),
      ],
      out_specs=pl.BlockSpec((N // 10, D), lambda i: (i, 0)),
      out_shape=jax.ShapeDtypeStruct((N, D), jnp.float32),
  )(feat, W, b.reshape(1, D))

  zeros = jnp.zeros((NPAD, D), jnp.float32)
  partials = _sc_aggregate(srcp, dstp, h, zeros)

  out = pl.pallas_call(
      _add_body,
      grid=(10,),
      in_specs=[
          pl.BlockSpec((N // 10, D), lambda i: (i, 0)),
          pl.BlockSpec((N // 10, D), lambda i: (i, 0)),
      ],
      out_specs=pl.BlockSpec((N // 10, D), lambda i: (i, 0)),
      out_shape=jax.ShapeDtypeStruct((N, D), jnp.float32),
  )(partials[0, :N], partials[1, :N])
  return out
